# baseline (speedup 1.0000x reference)
import functools

import jax
import jax.numpy as jnp
from jax import lax
from jax.experimental import pallas as pl
from jax.experimental.pallas import tpu as pltpu
from jax.experimental.pallas import tpu_sc as plsc

N = 100000
D = 128
B = 100
NW = 32
RPW = N // NW
CH = 125
NCH = RPW // CH
IDS_PAD = 100352
ACC = B * D


def _nrsqrt(x):
    i = lax.bitcast_convert_type(x, jnp.int32)
    i = jnp.int32(0x5F3759DF) - lax.shift_right_arithmetic(i, 1)
    y = lax.bitcast_convert_type(i, jnp.float32)
    for _ in range(3):
        y = y * (1.5 - 0.5 * x * y * y)
    return y


def _sc_body(z_hbm, ids_hbm, outS, outT, outC,
             zbuf0, zbuf1, idsbuf, accS, accT, accC, sem0, sem1, semi):
    cid = lax.axis_index("c")
    sid = lax.axis_index("s")
    wid = cid * 16 + sid
    row0 = wid * RPW
    astart = (row0 // 8) * 8
    off = row0 - astart

    ids_cp = pltpu.async_copy(ids_hbm.at[pl.ds(astart, 3136)], idsbuf, semi)

    zeros16 = jnp.zeros((16,), jnp.float32)

    def zero_body(i, _):
        accS[pl.ds(i * 16, 16)] = zeros16
        accT[pl.ds(i * 16, 16)] = zeros16
        return 0

    lax.fori_loop(0, ACC // 16, zero_body, 0)

    def zero_cnt(i, _):
        accC[pl.ds(i * 16, 16)] = zeros16
        return 0

    lax.fori_loop(0, 8, zero_cnt, 0)

    def start(c, buf, sem):
        return pltpu.async_copy(z_hbm.at[pl.ds(row0 + c * CH, CH)], buf, sem)

    def wait(buf, sem):
        pltpu.make_async_copy(z_hbm.at[pl.ds(row0, CH)], buf, sem).wait()

    ids_cp.wait()
    start(0, zbuf0, sem0)

    lanes = lax.iota(jnp.int32, 16)
    ones16 = jnp.ones((16,), jnp.float32)

    def process(zbuf, c):
        for g in range(8):
            if g < 7:
                msk = None
                lrow_in = g * 16 + lanes
            else:
                msk = lanes < 13
                lrow_in = jnp.minimum(g * 16 + lanes, 124)
            idv = plsc.load_gather(idsbuf, [off + c * CH + lrow_in])

            def p1(j, ss, lrow_in=lrow_in):
                for u in range(8):
                    d = j * 8 + u
                    dcol = jnp.full((16,), 0, jnp.int32) + d
                    v = plsc.load_gather(zbuf, [lrow_in, dcol])
                    ss = ss + v * v
                return ss

            ss = lax.fori_loop(0, 16, p1, jnp.zeros((16,), jnp.float32))
            rinv = jnp.where(ss >= 1e-16, _nrsqrt(ss), 1e8)
            sbase = idv * D

            def p2(j, _, lrow_in=lrow_in, sbase=sbase, rinv=rinv, msk=msk):
                for u in range(8):
                    d = j * 8 + u
                    dcol = jnp.full((16,), 0, jnp.int32) + d
                    v = plsc.load_gather(zbuf, [lrow_in, dcol])
                    plsc.addupdate_scatter(accS, [sbase + dcol], v, mask=msk)
                    plsc.addupdate_scatter(accT, [sbase + dcol], v * rinv,
                                           mask=msk)
                return 0

            lax.fori_loop(0, 16, p2, 0)
            plsc.addupdate_scatter(accC, [idv], ones16, mask=msk)

    def chunk_pair(i, _):
        c = i * 2
        start(c + 1, zbuf1, sem1)
        wait(zbuf0, sem0)
        process(zbuf0, c)
        start(c + 2, zbuf0, sem0)
        wait(zbuf1, sem1)
        process(zbuf1, c + 1)
        return 0

    lax.fori_loop(0, (NCH - 1) // 2, chunk_pair, 0)
    wait(zbuf0, sem0)
    process(zbuf0, NCH - 1)

    pltpu.sync_copy(accS, outS.at[wid])
    pltpu.sync_copy(accT, outT.at[wid])
    pltpu.sync_copy(accC, outC.at[wid])


_sc_call = functools.partial(
    pl.kernel,
    out_type=(
        jax.ShapeDtypeStruct((NW, ACC), jnp.float32),
        jax.ShapeDtypeStruct((NW, ACC), jnp.float32),
        jax.ShapeDtypeStruct((NW, D), jnp.float32),
    ),
    mesh=plsc.VectorSubcoreMesh(core_axis_name="c", subcore_axis_name="s"),
    compiler_params=pltpu.CompilerParams(
        use_tc_tiling_on_sc=False, needs_layout_passes=False),
    scratch_types=[
        pltpu.VMEM((CH, D), jnp.float32),
        pltpu.VMEM((CH, D), jnp.float32),
        pltpu.VMEM((3136,), jnp.int32),
        pltpu.VMEM((ACC,), jnp.float32),
        pltpu.VMEM((ACC,), jnp.float32),
        pltpu.VMEM((D,), jnp.float32),
        pltpu.SemaphoreType.DMA,
        pltpu.SemaphoreType.DMA,
        pltpu.SemaphoreType.DMA,
    ],
)(_sc_body)


def _tc_epilogue(pS_ref, pT_ref, pC_ref, out_ref):
    S = jnp.sum(pS_ref[...], axis=0)
    T = jnp.sum(pT_ref[...], axis=0)
    cnt = jnp.sum(pC_ref[...], axis=0).reshape(D, 1)[:B]
    cntc = jnp.maximum(cnt, 1.0)
    c = S / cntc
    dot = jnp.sum(T * c, axis=1, keepdims=True)
    cn = jnp.maximum(jnp.sqrt(jnp.sum(c * c, axis=1, keepdims=True)), 1e-8)
    cos_mean = dot / (cn * cntc)
    valid = cnt > 1.0
    per = jnp.where(valid, 1.0 - cos_mean, 0.0)
    nv = jnp.sum(valid.astype(jnp.float32))
    out_ref[0, 0] = jnp.sum(per) / jnp.maximum(nv, 1.0)


def kernel(z, poi_to_road_block):
    ids = poi_to_road_block.astype(jnp.int32)
    ids_pad = jnp.concatenate(
        [ids, jnp.zeros((IDS_PAD - N,), jnp.int32)])
    pS, pT, pC = _sc_call(z, ids_pad)
    pS3 = pS.reshape(NW, B, D)
    pT3 = pT.reshape(NW, B, D)
    loss = pl.pallas_call(
        _tc_epilogue,
        out_shape=jax.ShapeDtypeStruct((1, 1), jnp.float32),
        out_specs=pl.BlockSpec(memory_space=pltpu.SMEM),
    )(pS3, pT3, pC)
    return loss[0, 0]


# speedup vs baseline: 4.5233x; 4.5233x over previous
import functools

import jax
import jax.numpy as jnp
from jax import lax
from jax.experimental import pallas as pl
from jax.experimental.pallas import tpu as pltpu
from jax.experimental.pallas import tpu_sc as plsc

N = 100000
D = 128
B = 100
NW = 32
RPW = N // NW
CH = 125
NCH = RPW // CH
IDS_PAD = 100352
ACC = B * D
CHW = CH * D


def _nrsqrt_scalar(x):
    i = lax.bitcast_convert_type(x, jnp.int32)
    i = jnp.int32(0x5F3759DF) - lax.shift_right_arithmetic(i, 1)
    y = lax.bitcast_convert_type(i, jnp.float32)
    for _ in range(3):
        y = y * (1.5 - 0.5 * x * y * y)
    return y


def _sc_body(z_hbm, ids_hbm, outS, outT, outC,
             zbuf0, zbuf1, idsbuf, accS, accT, accC, sem0, sem1, semi):
    cid = lax.axis_index("c")
    sid = lax.axis_index("s")
    wid = cid * 16 + sid
    row0 = wid * RPW
    astart = (row0 // 8) * 8
    off = row0 - astart

    ids_cp = pltpu.async_copy(ids_hbm.at[pl.ds(astart, 3136)], idsbuf, semi)

    zeros16 = jnp.zeros((16,), jnp.float32)

    def zero_body(i, _):
        accS[pl.ds(i * 16, 16)] = zeros16
        accT[pl.ds(i * 16, 16)] = zeros16
        return 0

    lax.fori_loop(0, ACC // 16, zero_body, 0)

    def zero_cnt(i, _):
        accC[pl.ds(i * 16, 16)] = zeros16
        return 0

    lax.fori_loop(0, 8, zero_cnt, 0)

    def start(c, buf, sem):
        return pltpu.async_copy(
            z_hbm.at[pl.ds((row0 + c * CH) * D, CHW)], buf, sem)

    def wait(buf, sem):
        pltpu.make_async_copy(z_hbm.at[pl.ds(row0 * D, CHW)], buf, sem).wait()

    ids_cp.wait()
    start(0, zbuf0, sem0)

    lanes = lax.iota(jnp.int32, 16)
    ones16 = jnp.ones((16,), jnp.float32)
    tailmask = lanes < 13

    def do_rows(zbuf, idv16, gbase, nrows):
        for r in range(nrows):
            idr = idv16[r]
            rb = (gbase + r) * D
            sb = idr * D
            v = [zbuf[pl.ds(rb + k * 16, 16)] for k in range(8)]
            ss = v[0] * v[0]
            for k in range(1, 8):
                ss = ss + v[k] * v[k]
            tot = jnp.sum(ss)
            rinv = jnp.where(tot >= 1e-16, _nrsqrt_scalar(tot),
                             jnp.float32(1e8))
            rv = jnp.full((16,), rinv, jnp.float32)
            for k in range(8):
                plsc.addupdate(accS.at[pl.ds(sb + k * 16, 16)], v[k])
                plsc.addupdate(accT.at[pl.ds(sb + k * 16, 16)], v[k] * rv)

    def process(zbuf, c):
        ib = off + c * CH

        def grp(g, _):
            idv16 = idsbuf[pl.ds(ib + g * 16, 16)]
            plsc.addupdate_scatter(accC, [idv16], ones16)
            do_rows(zbuf, idv16, g * 16, 16)
            return 0

        lax.fori_loop(0, 7, grp, 0)
        idv16 = idsbuf[pl.ds(ib + 112, 16)]
        plsc.addupdate_scatter(accC, [idv16], ones16, mask=tailmask)
        do_rows(zbuf, idv16, 112, 13)

    def chunk_pair(i, _):
        c = i * 2
        start(c + 1, zbuf1, sem1)
        wait(zbuf0, sem0)
        process(zbuf0, c)
        start(c + 2, zbuf0, sem0)
        wait(zbuf1, sem1)
        process(zbuf1, c + 1)
        return 0

    lax.fori_loop(0, (NCH - 1) // 2, chunk_pair, 0)
    wait(zbuf0, sem0)
    process(zbuf0, NCH - 1)

    pltpu.sync_copy(accS, outS.at[wid])
    pltpu.sync_copy(accT, outT.at[wid])
    pltpu.sync_copy(accC, outC.at[wid])


_sc_call = functools.partial(
    pl.kernel,
    out_type=(
        jax.ShapeDtypeStruct((NW, ACC), jnp.float32),
        jax.ShapeDtypeStruct((NW, ACC), jnp.float32),
        jax.ShapeDtypeStruct((NW, D), jnp.float32),
    ),
    mesh=plsc.VectorSubcoreMesh(core_axis_name="c", subcore_axis_name="s"),
    compiler_params=pltpu.CompilerParams(
        use_tc_tiling_on_sc=False, needs_layout_passes=False),
    scratch_types=[
        pltpu.VMEM((CHW,), jnp.float32),
        pltpu.VMEM((CHW,), jnp.float32),
        pltpu.VMEM((3136,), jnp.int32),
        pltpu.VMEM((ACC,), jnp.float32),
        pltpu.VMEM((ACC,), jnp.float32),
        pltpu.VMEM((D,), jnp.float32),
        pltpu.SemaphoreType.DMA,
        pltpu.SemaphoreType.DMA,
        pltpu.SemaphoreType.DMA,
    ],
)(_sc_body)


def _tc_epilogue(pS_ref, pT_ref, pC_ref, out_ref):
    S = jnp.sum(pS_ref[...], axis=0)
    T = jnp.sum(pT_ref[...], axis=0)
    cnt = jnp.sum(pC_ref[...], axis=0).reshape(D, 1)[:B]
    cntc = jnp.maximum(cnt, 1.0)
    c = S / cntc
    dot = jnp.sum(T * c, axis=1, keepdims=True)
    cn = jnp.maximum(jnp.sqrt(jnp.sum(c * c, axis=1, keepdims=True)), 1e-8)
    cos_mean = dot / (cn * cntc)
    valid = cnt > 1.0
    per = jnp.where(valid, 1.0 - cos_mean, 0.0)
    nv = jnp.sum(valid.astype(jnp.float32))
    out_ref[0, 0] = jnp.sum(per) / jnp.maximum(nv, 1.0)


def kernel(z, poi_to_road_block):
    ids = poi_to_road_block.astype(jnp.int32)
    ids_pad = jnp.concatenate(
        [ids, jnp.zeros((IDS_PAD - N,), jnp.int32)])
    pS, pT, pC = _sc_call(z.reshape(-1), ids_pad)
    pS3 = pS.reshape(NW, B, D)
    pT3 = pT.reshape(NW, B, D)
    loss = pl.pallas_call(
        _tc_epilogue,
        out_shape=jax.ShapeDtypeStruct((1, 1), jnp.float32),
        out_specs=pl.BlockSpec(memory_space=pltpu.SMEM),
    )(pS3, pT3, pC)
    return loss[0, 0]


# speedup vs baseline: 7.0763x; 1.5644x over previous
import functools

import jax
import jax.numpy as jnp
from jax import lax
from jax.experimental import pallas as pl
from jax.experimental.pallas import tpu as pltpu
from jax.experimental.pallas import tpu_sc as plsc

N = 100000
D = 128
B = 100
NW = 32
RPW = N // NW
CH = 125
NCH = RPW // CH
IDS_PAD = 100352
ACC = B * D
CHW = CH * D


def _nrsqrt(x):
    i = lax.bitcast_convert_type(x, jnp.int32)
    i = jnp.int32(0x5F3759DF) - lax.shift_right_arithmetic(i, 1)
    y = lax.bitcast_convert_type(i, jnp.float32)
    for _ in range(3):
        y = y * (1.5 - 0.5 * x * y * y)
    return y


_BCAST_DNUMS = lax.GatherDimensionNumbers(
    offset_dims=(), collapsed_slice_dims=(0,), start_index_map=(0,))
_LANE15 = None


def _bcast_last(x):
    idx = jnp.full((16, 1), 15, jnp.int32)
    return lax.gather(x, idx, _BCAST_DNUMS, (1,),
                      mode=lax.GatherScatterMode.PROMISE_IN_BOUNDS)


def _sc_body(z_hbm, ids_hbm, outS, outT, outC,
             zbuf0, zbuf1, idsbuf, accS, accT, accC, sem0, sem1, semi):
    cid = lax.axis_index("c")
    sid = lax.axis_index("s")
    wid = cid * 16 + sid
    row0 = wid * RPW
    astart = (row0 // 8) * 8
    off = row0 - astart

    ids_cp = pltpu.async_copy(ids_hbm.at[pl.ds(astart, 3136)], idsbuf, semi)

    zeros16 = jnp.zeros((16,), jnp.float32)

    def zero_body(i, _):
        accS[pl.ds(i * 16, 16)] = zeros16
        accT[pl.ds(i * 16, 16)] = zeros16
        return 0

    lax.fori_loop(0, ACC // 16, zero_body, 0)

    def zero_cnt(i, _):
        accC[pl.ds(i * 16, 16)] = zeros16
        return 0

    lax.fori_loop(0, 8, zero_cnt, 0)

    def start(c, buf, sem):
        return pltpu.async_copy(
            z_hbm.at[pl.ds((row0 + c * CH) * D, CHW)], buf, sem)

    def wait(buf, sem):
        pltpu.make_async_copy(z_hbm.at[pl.ds(row0 * D, CHW)], buf, sem).wait()

    ids_cp.wait()
    start(0, zbuf0, sem0)

    lanes = lax.iota(jnp.int32, 16)
    ones16 = jnp.ones((16,), jnp.float32)
    tailmask = lanes < 13

    def row_load(zbuf, gbase, r):
        rb = (gbase + r) * D
        v = [zbuf[pl.ds(rb + k * 16, 16)] for k in range(8)]
        ss = v[0] * v[0]
        for k in range(1, 8):
            ss = ss + v[k] * v[k]
        return v, ss

    def row_rinv(ss):
        tot = _bcast_last(plsc.cumsum(ss))
        return jnp.where(tot >= 1e-16, _nrsqrt(tot), jnp.float32(1e8))

    def row_store(idv16, r, v, rv):
        sb = idv16[r] * D
        for k in range(8):
            plsc.addupdate(accS.at[pl.ds(sb + k * 16, 16)], v[k])
            plsc.addupdate(accT.at[pl.ds(sb + k * 16, 16)], v[k] * rv)

    def do_rows(zbuf, idv16, gbase, nrows):
        for r in range(0, nrows - 1, 2):
            va, ssa = row_load(zbuf, gbase, r)
            vb, ssb = row_load(zbuf, gbase, r + 1)
            ra = row_rinv(ssa)
            rb = row_rinv(ssb)
            row_store(idv16, r, va, ra)
            row_store(idv16, r + 1, vb, rb)
        if nrows % 2:
            v, ss = row_load(zbuf, gbase, nrows - 1)
            rv = row_rinv(ss)
            row_store(idv16, nrows - 1, v, rv)

    def process(zbuf, c):
        ib = off + c * CH

        def grp(g, _):
            idv16 = idsbuf[pl.ds(ib + g * 16, 16)]
            plsc.addupdate_scatter(accC, [idv16], ones16)
            do_rows(zbuf, idv16, g * 16, 16)
            return 0

        lax.fori_loop(0, 7, grp, 0)
        idv16 = idsbuf[pl.ds(ib + 112, 16)]
        plsc.addupdate_scatter(accC, [idv16], ones16, mask=tailmask)
        do_rows(zbuf, idv16, 112, 13)

    def chunk_pair(i, _):
        c = i * 2
        start(c + 1, zbuf1, sem1)
        wait(zbuf0, sem0)
        process(zbuf0, c)
        start(c + 2, zbuf0, sem0)
        wait(zbuf1, sem1)
        process(zbuf1, c + 1)
        return 0

    lax.fori_loop(0, (NCH - 1) // 2, chunk_pair, 0)
    wait(zbuf0, sem0)
    process(zbuf0, NCH - 1)

    pltpu.sync_copy(accS, outS.at[wid])
    pltpu.sync_copy(accT, outT.at[wid])
    pltpu.sync_copy(accC, outC.at[wid])


_sc_call = functools.partial(
    pl.kernel,
    out_type=(
        jax.ShapeDtypeStruct((NW, ACC), jnp.float32),
        jax.ShapeDtypeStruct((NW, ACC), jnp.float32),
        jax.ShapeDtypeStruct((NW, D), jnp.float32),
    ),
    mesh=plsc.VectorSubcoreMesh(core_axis_name="c", subcore_axis_name="s"),
    compiler_params=pltpu.CompilerParams(
        use_tc_tiling_on_sc=False, needs_layout_passes=False),
    scratch_types=[
        pltpu.VMEM((CHW,), jnp.float32),
        pltpu.VMEM((CHW,), jnp.float32),
        pltpu.VMEM((3136,), jnp.int32),
        pltpu.VMEM((ACC,), jnp.float32),
        pltpu.VMEM((ACC,), jnp.float32),
        pltpu.VMEM((D,), jnp.float32),
        pltpu.SemaphoreType.DMA,
        pltpu.SemaphoreType.DMA,
        pltpu.SemaphoreType.DMA,
    ],
)(_sc_body)


def _tc_epilogue(pS_ref, pT_ref, pC_ref, out_ref):
    S = jnp.sum(pS_ref[...], axis=0)
    T = jnp.sum(pT_ref[...], axis=0)
    cnt = jnp.sum(pC_ref[...], axis=0).reshape(D, 1)[:B]
    cntc = jnp.maximum(cnt, 1.0)
    c = S / cntc
    dot = jnp.sum(T * c, axis=1, keepdims=True)
    cn = jnp.maximum(jnp.sqrt(jnp.sum(c * c, axis=1, keepdims=True)), 1e-8)
    cos_mean = dot / (cn * cntc)
    valid = cnt > 1.0
    per = jnp.where(valid, 1.0 - cos_mean, 0.0)
    nv = jnp.sum(valid.astype(jnp.float32))
    out_ref[0, 0] = jnp.sum(per) / jnp.maximum(nv, 1.0)


def kernel(z, poi_to_road_block):
    ids = poi_to_road_block.astype(jnp.int32)
    ids_pad = jnp.concatenate(
        [ids, jnp.zeros((IDS_PAD - N,), jnp.int32)])
    pS, pT, pC = _sc_call(z.reshape(-1), ids_pad)
    pS3 = pS.reshape(NW, B, D)
    pT3 = pT.reshape(NW, B, D)
    loss = pl.pallas_call(
        _tc_epilogue,
        out_shape=jax.ShapeDtypeStruct((1, 1), jnp.float32),
        out_specs=pl.BlockSpec(memory_space=pltpu.SMEM),
    )(pS3, pT3, pC)
    return loss[0, 0]


# speedup vs baseline: 10.3665x; 1.4650x over previous
import functools

import jax
import jax.numpy as jnp
from jax import lax
from jax.experimental import pallas as pl
from jax.experimental.pallas import tpu as pltpu
from jax.experimental.pallas import tpu_sc as plsc

N = 100000
D = 128
B = 100
NW = 32
RPW = N // NW
CH = 125
NCH = RPW // CH
IDS_PAD = 100352
ACC = B * D
CHW = CH * D


def _nrsqrt(x):
    i = lax.bitcast_convert_type(x, jnp.int32)
    i = jnp.int32(0x5F3759DF) - lax.shift_right_arithmetic(i, 1)
    y = lax.bitcast_convert_type(i, jnp.float32)
    for _ in range(3):
        y = y * (1.5 - 0.5 * x * y * y)
    return y


_BCAST_DNUMS = lax.GatherDimensionNumbers(
    offset_dims=(), collapsed_slice_dims=(0,), start_index_map=(0,))
_LANE15 = None


def _bcast_last(x):
    idx = jnp.full((16, 1), 15, jnp.int32)
    return lax.gather(x, idx, _BCAST_DNUMS, (1,),
                      mode=lax.GatherScatterMode.PROMISE_IN_BOUNDS)


def _sc_body(z_hbm, ids_hbm, outS, outT, outC,
             zbuf0, zbuf1, idsbuf, accS, accT, accC, sem0, sem1, semi):
    cid = lax.axis_index("c")
    sid = lax.axis_index("s")
    wid = cid * 16 + sid
    row0 = wid * RPW
    astart = (row0 // 8) * 8
    off = row0 - astart

    ids_cp = pltpu.async_copy(ids_hbm.at[pl.ds(astart, 3136)], idsbuf, semi)

    zeros16 = jnp.zeros((16,), jnp.float32)

    def zero_body(i, _):
        accS[pl.ds(i * 16, 16)] = zeros16
        accT[pl.ds(i * 16, 16)] = zeros16
        return 0

    lax.fori_loop(0, ACC // 16, zero_body, 0)

    def zero_cnt(i, _):
        accC[pl.ds(i * 16, 16)] = zeros16
        return 0

    lax.fori_loop(0, 8, zero_cnt, 0)

    def start(c, buf, sem):
        return pltpu.async_copy(
            z_hbm.at[pl.ds((row0 + c * CH) * D, CHW)], buf, sem)

    def wait(buf, sem):
        pltpu.make_async_copy(z_hbm.at[pl.ds(row0 * D, CHW)], buf, sem).wait()

    ids_cp.wait()
    start(0, zbuf0, sem0)

    lanes = lax.iota(jnp.int32, 16)
    ones16 = jnp.ones((16,), jnp.float32)
    tailmask = lanes < 13

    def row_load(zbuf, gbase, r):
        rb = (gbase + r) * D
        v = [zbuf[pl.ds(rb + k * 16, 16)] for k in range(8)]
        ss = v[0] * v[0]
        for k in range(1, 8):
            ss = ss + v[k] * v[k]
        return v, ss

    def row_rinv(ss):
        tot = _bcast_last(plsc.cumsum(ss))
        return jnp.where(tot >= 1e-16, _nrsqrt(tot), jnp.float32(1e8))

    def row_store(idv16, r, v, rv):
        sb = idv16[r] * D
        for k in range(8):
            plsc.addupdate(accS.at[pl.ds(sb + k * 16, 16)], v[k])
            plsc.addupdate(accT.at[pl.ds(sb + k * 16, 16)], v[k] * rv)

    def do_rows(zbuf, idv16, gbase, nrows):
        starts = list(range(0, nrows - 1, 2))
        if nrows % 2:
            starts.append(nrows - 1)
        prev = None
        for a in starts:
            cur = [(a,) + row_load(zbuf, gbase, a)]
            if a + 1 < nrows:
                cur.append((a + 1,) + row_load(zbuf, gbase, a + 1))
            cur = [(r, v, row_rinv(ss)) for (r, v, ss) in cur]
            if prev is not None:
                for (r, v, rv) in prev:
                    row_store(idv16, r, v, rv)
            prev = cur
        for (r, v, rv) in prev:
            row_store(idv16, r, v, rv)

    def process(zbuf, c):
        ib = off + c * CH

        def grp(g, _):
            idv16 = idsbuf[pl.ds(ib + g * 16, 16)]
            plsc.addupdate_scatter(accC, [idv16], ones16)
            do_rows(zbuf, idv16, g * 16, 16)
            return 0

        lax.fori_loop(0, 7, grp, 0)
        idv16 = idsbuf[pl.ds(ib + 112, 16)]
        plsc.addupdate_scatter(accC, [idv16], ones16, mask=tailmask)
        do_rows(zbuf, idv16, 112, 13)

    def chunk_pair(i, _):
        c = i * 2
        start(c + 1, zbuf1, sem1)
        wait(zbuf0, sem0)
        process(zbuf0, c)
        start(c + 2, zbuf0, sem0)
        wait(zbuf1, sem1)
        process(zbuf1, c + 1)
        return 0

    lax.fori_loop(0, (NCH - 1) // 2, chunk_pair, 0)
    wait(zbuf0, sem0)
    process(zbuf0, NCH - 1)

    pltpu.sync_copy(accS, outS.at[wid])
    pltpu.sync_copy(accT, outT.at[wid])
    pltpu.sync_copy(accC, outC.at[wid])


_sc_call = functools.partial(
    pl.kernel,
    out_type=(
        jax.ShapeDtypeStruct((NW, ACC), jnp.float32),
        jax.ShapeDtypeStruct((NW, ACC), jnp.float32),
        jax.ShapeDtypeStruct((NW, D), jnp.float32),
    ),
    mesh=plsc.VectorSubcoreMesh(core_axis_name="c", subcore_axis_name="s"),
    compiler_params=pltpu.CompilerParams(
        use_tc_tiling_on_sc=False, needs_layout_passes=False),
    scratch_types=[
        pltpu.VMEM((CHW,), jnp.float32),
        pltpu.VMEM((CHW,), jnp.float32),
        pltpu.VMEM((3136,), jnp.int32),
        pltpu.VMEM((ACC,), jnp.float32),
        pltpu.VMEM((ACC,), jnp.float32),
        pltpu.VMEM((D,), jnp.float32),
        pltpu.SemaphoreType.DMA,
        pltpu.SemaphoreType.DMA,
        pltpu.SemaphoreType.DMA,
    ],
)(_sc_body)


def _tc_epilogue(pS_ref, pT_ref, pC_ref, out_ref):
    S = jnp.sum(pS_ref[...], axis=0)
    T = jnp.sum(pT_ref[...], axis=0)
    cnt = jnp.sum(pC_ref[...], axis=0).reshape(D, 1)[:B]
    cntc = jnp.maximum(cnt, 1.0)
    c = S / cntc
    dot = jnp.sum(T * c, axis=1, keepdims=True)
    cn = jnp.maximum(jnp.sqrt(jnp.sum(c * c, axis=1, keepdims=True)), 1e-8)
    cos_mean = dot / (cn * cntc)
    valid = cnt > 1.0
    per = jnp.where(valid, 1.0 - cos_mean, 0.0)
    nv = jnp.sum(valid.astype(jnp.float32))
    out_ref[0, 0] = jnp.sum(per) / jnp.maximum(nv, 1.0)


def kernel(z, poi_to_road_block):
    ids = poi_to_road_block.astype(jnp.int32)
    ids_pad = jnp.concatenate(
        [ids, jnp.zeros((IDS_PAD - N,), jnp.int32)])
    pS, pT, pC = _sc_call(z.reshape(-1), ids_pad)
    pS3 = pS.reshape(NW, B, D)
    pT3 = pT.reshape(NW, B, D)
    loss = pl.pallas_call(
        _tc_epilogue,
        out_shape=jax.ShapeDtypeStruct((1, 1), jnp.float32),
        out_specs=pl.BlockSpec(memory_space=pltpu.SMEM),
    )(pS3, pT3, pC)
    return loss[0, 0]
